# baseline (speedup 1.0000x reference)
import jax
import jax.numpy as jnp
from jax.experimental import pallas as pl
from jax.experimental.pallas import tpu as pltpu

_BS = 4
_MAX_SEQLEN = 128
_LAYERS = 6
_HEADS = 32
_HEAD_DIM = 128
_STRIDE = 16
_NUM_PAGES = 64
_FEAT = _HEADS * _HEAD_DIM
_ROWS = _LAYERS * 2 * _STRIDE
_H_ROWS = _BS * _MAX_SEQLEN
_H_BLOCK = _H_ROWS // _NUM_PAGES


def _layer_consts():
    x = jnp.float32(1.0)
    ks, vs = [], []
    for _ in range(_LAYERS):
        xk = x * jnp.float32(2.0)
        xv = x * jnp.float32(4.0)
        ks.append(xk)
        vs.append(xv)
        x = x + x * xk * xv
    return ks, vs, x


def _body(mask_ref, in_ref, out_ref, h_ref):
    p = pl.program_id(0)
    ks, vs, h_final = _layer_consts()

    r = jax.lax.broadcasted_iota(jnp.int32, (_ROWS, _FEAT), 0)
    layer_idx = r // (2 * _STRIDE)
    kv = (r // _STRIDE) % 2
    pat = jnp.zeros((_ROWS, _FEAT), jnp.float32)
    for l in range(_LAYERS):
        pat = jnp.where(layer_idx == l, jnp.where(kv == 0, ks[l], vs[l]), pat)

    member = mask_ref[p] > 0
    out_ref[...] = jnp.where(member, pat[None], in_ref[...])
    h_ref[...] = jnp.full((_H_BLOCK, _FEAT), h_final)


def kernel(seq_lens, attn_block_ids, attn_page_slab):
    del seq_lens
    ids = attn_block_ids.reshape(-1).astype(jnp.int32)
    mask = jnp.zeros((_NUM_PAGES,), jnp.int32).at[ids].set(1)
    slab = attn_page_slab.reshape(_NUM_PAGES, _ROWS, _FEAT)

    out, h = pl.pallas_call(
        _body,
        grid=(_NUM_PAGES,),
        in_specs=[
            pl.BlockSpec(memory_space=pltpu.SMEM),
            pl.BlockSpec((1, _ROWS, _FEAT), lambda i: (i, 0, 0)),
        ],
        out_specs=[
            pl.BlockSpec((1, _ROWS, _FEAT), lambda i: (i, 0, 0)),
            pl.BlockSpec((_H_BLOCK, _FEAT), lambda i: (i, 0)),
        ],
        out_shape=[
            jax.ShapeDtypeStruct((_NUM_PAGES, _ROWS, _FEAT), jnp.float32),
            jax.ShapeDtypeStruct((_H_ROWS, _FEAT), jnp.float32),
        ],
        compiler_params=pltpu.CompilerParams(
            dimension_semantics=("arbitrary",),
        ),
    )(mask, slab)

    h = h.reshape(_BS, _MAX_SEQLEN, _FEAT)
    slab_out = out.reshape(_NUM_PAGES, _LAYERS, 2, _STRIDE, _HEADS, _HEAD_DIM)
    return h, slab_out
